# baseline (device time: 10550 ns/iter reference)
import jax
import jax.numpy as jnp
from jax import lax
from jax.experimental import pallas as pl
from jax.experimental.pallas import tpu as pltpu

N_DEV = 4
E_LOCAL = 2
N_EXP = 8
N_TOK = 256
D_IN = 128
D_OUT = 256
NC = 4
ROWS = N_TOK // NC


def kernel(x, router_W, route_idx, expert_W):
    def body(x_ref, rw_ref, idx_ref, ew_ref, out_ref,
             comm_ref, sbuf_ref, send_sems, recv_sems):
        my_pos = lax.axis_index("i")
        partners = (jnp.bitwise_xor(my_pos, 1), 3 - my_pos)

        barrier_sem = pltpu.get_barrier_semaphore()
        for s in range(2):
            pl.semaphore_signal(
                barrier_sem, inc=1,
                device_id=(partners[s],), device_id_type=pl.DeviceIdType.MESH,
            )

        xv = x_ref[:, :]

        scores = jnp.dot(xv, rw_ref[:, :], preferred_element_type=jnp.float32)
        s_max = jnp.max(scores, axis=1, keepdims=True)
        probs = jnp.exp(scores - s_max)

        idx0 = idx_ref[:, 0:1]
        idx1 = idx_ref[:, 1:2]
        iota = lax.broadcasted_iota(jnp.int32, (N_TOK, N_EXP), 1)
        p0 = jnp.sum(jnp.where(iota == idx0, probs, 0.0), axis=1, keepdims=True)
        p1 = jnp.sum(jnp.where(iota == idx1, probs, 0.0), axis=1, keepdims=True)
        gate_sum = p0 + p1

        gates = []
        for l in range(E_LOCAL):
            e_g = my_pos * E_LOCAL + l
            sel = (idx0 == e_g) | (idx1 == e_g)
            p_e = jnp.sum(jnp.where(iota == e_g, probs, 0.0),
                          axis=1, keepdims=True)
            gates.append(
                jnp.where(sel, p_e / gate_sum, 0.0).astype(jnp.bfloat16))

        def chunk_rdma(s, c):
            return pltpu.make_async_remote_copy(
                src_ref=sbuf_ref.at[s, c],
                dst_ref=comm_ref.at[s, c],
                send_sem=send_sems.at[s, c],
                recv_sem=recv_sems.at[s, c],
                device_id=(partners[(s + c) % 2],),
                device_id_type=pl.DeviceIdType.MESH,
            )

        x_b = xv.astype(jnp.bfloat16)
        ew_b = [ew_ref[l].astype(jnp.bfloat16) for l in range(E_LOCAL)]
        step1 = []
        for c in range(NC):
            r = slice(c * ROWS, (c + 1) * ROWS)
            x_c = x_b[r]
            part = (jnp.dot(gates[0][r] * x_c, ew_b[0],
                            preferred_element_type=jnp.float32)
                    + jnp.dot(gates[1][r] * x_c, ew_b[1],
                              preferred_element_type=jnp.float32))
            out_ref[r, :] = part
            sbuf_ref[0, c] = part.astype(jnp.bfloat16)
            if c == 0:
                pl.semaphore_wait(barrier_sem, 2)
            rdma = chunk_rdma(0, c)
            rdma.start()
            step1.append(rdma)

        step2 = []
        for c in range(NC):
            r = slice(c * ROWS, (c + 1) * ROWS)
            step1[c].wait()
            acc = out_ref[r, :] + comm_ref[0, c].astype(jnp.float32)
            out_ref[r, :] = acc
            sbuf_ref[1, c] = acc.astype(jnp.bfloat16)
            rdma = chunk_rdma(1, c)
            rdma.start()
            step2.append(rdma)

        for c in range(NC):
            r = slice(c * ROWS, (c + 1) * ROWS)
            step2[c].wait()
            out_ref[r, :] += comm_ref[1, c].astype(jnp.float32)

    return pl.pallas_call(
        body,
        out_shape=jax.ShapeDtypeStruct((N_TOK, D_OUT), jnp.float32),
        in_specs=[pl.BlockSpec(memory_space=pltpu.VMEM)] * 4,
        out_specs=pl.BlockSpec(memory_space=pltpu.VMEM),
        scratch_shapes=[
            pltpu.VMEM((2, NC, ROWS, D_OUT), jnp.bfloat16),
            pltpu.VMEM((2, NC, ROWS, D_OUT), jnp.bfloat16),
            pltpu.SemaphoreType.DMA((2, NC)),
            pltpu.SemaphoreType.DMA((2, NC)),
        ],
        compiler_params=pltpu.CompilerParams(collective_id=0),
    )(x, router_W, route_idx, expert_W)


# device time: 10303 ns/iter; 1.0240x vs baseline; 1.0240x over previous
import jax
import jax.numpy as jnp
from jax import lax
from jax.experimental import pallas as pl
from jax.experimental.pallas import tpu as pltpu

N_DEV = 4
E_LOCAL = 2
N_EXP = 8
N_TOK = 256
D_IN = 128
D_OUT = 256
NC = 8
ROWS = N_TOK // NC


def kernel(x, router_W, route_idx, expert_W):
    def body(x_ref, rw_ref, idx_ref, ew_ref, out_ref,
             comm_ref, sbuf_ref, send_sems, recv_sems):
        my_pos = lax.axis_index("i")
        partners = (jnp.bitwise_xor(my_pos, 1), 3 - my_pos)

        barrier_sem = pltpu.get_barrier_semaphore()
        for s in range(2):
            pl.semaphore_signal(
                barrier_sem, inc=1,
                device_id=(partners[s],), device_id_type=pl.DeviceIdType.MESH,
            )

        xv = x_ref[:, :]

        scores = jnp.dot(xv, rw_ref[:, :], preferred_element_type=jnp.float32)
        s_max = jnp.max(scores, axis=1, keepdims=True)
        p = jnp.exp(scores - s_max)
        probs = p / jnp.sum(p, axis=1, keepdims=True)

        idx0 = idx_ref[:, 0:1]
        idx1 = idx_ref[:, 1:2]
        iota = lax.broadcasted_iota(jnp.int32, (N_TOK, N_EXP), 1)
        p0 = jnp.sum(jnp.where(iota == idx0, probs, 0.0), axis=1, keepdims=True)
        p1 = jnp.sum(jnp.where(iota == idx1, probs, 0.0), axis=1, keepdims=True)
        gate_sum = p0 + p1

        gates = []
        for l in range(E_LOCAL):
            e_g = my_pos * E_LOCAL + l
            sel = (idx0 == e_g) | (idx1 == e_g)
            p_e = jnp.sum(jnp.where(iota == e_g, probs, 0.0),
                          axis=1, keepdims=True)
            gates.append(jnp.where(sel, p_e / gate_sum, 0.0))

        def chunk_rdma(s, c):
            return pltpu.make_async_remote_copy(
                src_ref=sbuf_ref.at[s, c],
                dst_ref=comm_ref.at[s, c],
                send_sem=send_sems.at[s, c],
                recv_sem=recv_sems.at[s, c],
                device_id=(partners[(s + c) % 2],),
                device_id_type=pl.DeviceIdType.MESH,
            )

        x_b = xv.astype(jnp.bfloat16)
        ew_b = [ew_ref[l].astype(jnp.bfloat16) for l in range(E_LOCAL)]
        step1 = []
        for c in range(NC):
            r = slice(c * ROWS, (c + 1) * ROWS)
            x_c = x_b[r]
            part = (gates[0][r] * jnp.dot(x_c, ew_b[0],
                                          preferred_element_type=jnp.float32)
                    + gates[1][r] * jnp.dot(x_c, ew_b[1],
                                            preferred_element_type=jnp.float32))
            out_ref[r, :] = part
            sbuf_ref[0, c] = part.astype(jnp.bfloat16)
            if c == 0:
                pl.semaphore_wait(barrier_sem, 2)
            rdma = chunk_rdma(0, c)
            rdma.start()
            step1.append(rdma)

        step2 = []
        for c in range(NC):
            r = slice(c * ROWS, (c + 1) * ROWS)
            step1[c].wait()
            acc = out_ref[r, :] + comm_ref[0, c].astype(jnp.float32)
            out_ref[r, :] = acc
            sbuf_ref[1, c] = acc.astype(jnp.bfloat16)
            rdma = chunk_rdma(1, c)
            rdma.start()
            step2.append(rdma)

        for c in range(NC):
            r = slice(c * ROWS, (c + 1) * ROWS)
            step2[c].wait()
            out_ref[r, :] += comm_ref[1, c].astype(jnp.float32)

    return pl.pallas_call(
        body,
        out_shape=jax.ShapeDtypeStruct((N_TOK, D_OUT), jnp.float32),
        in_specs=[pl.BlockSpec(memory_space=pltpu.VMEM)] * 4,
        out_specs=pl.BlockSpec(memory_space=pltpu.VMEM),
        scratch_shapes=[
            pltpu.VMEM((2, NC, ROWS, D_OUT), jnp.bfloat16),
            pltpu.VMEM((2, NC, ROWS, D_OUT), jnp.bfloat16),
            pltpu.SemaphoreType.DMA((2, NC)),
            pltpu.SemaphoreType.DMA((2, NC)),
        ],
        compiler_params=pltpu.CompilerParams(collective_id=0),
    )(x, router_W, route_idx, expert_W)
